# baseline (speedup 1.0000x reference)
import functools

import jax
import jax.numpy as jnp
from jax import lax
from jax.experimental import pallas as pl
from jax.experimental.pallas import tpu as pltpu
from jax.experimental.pallas import tpu_sc as plsc


def _connector_sc(x_flat, indices, *, n_rows, n_idx, d):
    num_workers = 32
    rows_per_batch = n_rows // num_workers
    mesh = plsc.VectorSubcoreMesh(core_axis_name="c", subcore_axis_name="s")

    @functools.partial(
        pl.kernel,
        mesh=mesh,
        out_type=jax.ShapeDtypeStruct((num_workers * n_idx, d), jnp.float32),
        scratch_types=[
            pltpu.VMEM((n_idx,), jnp.int32),
            pltpu.SemaphoreType.DMA,
        ],
    )
    def k(x_hbm, idx_hbm, out_hbm, idx_s, sem):
        wid = lax.axis_index("s") * 2 + lax.axis_index("c")
        pltpu.sync_copy(idx_hbm, idx_s)
        row_base = wid * rows_per_batch
        out_base = wid * n_idx

        for j in range(n_idx):
            v = idx_s[pl.ds((j // 16) * 16, 16)]
            row = v[j % 16] + row_base
            pltpu.async_copy(
                x_hbm.at[pl.ds(row, 1)],
                out_hbm.at[pl.ds(out_base + j, 1)],
                sem,
            )

        @pl.loop(0, n_idx)
        def _(j):
            pltpu.make_async_copy(
                x_hbm.at[pl.ds(0, 1)],
                out_hbm.at[pl.ds(out_base, 1)],
                sem,
            ).wait()

    return k(x_flat, indices)


def kernel(x, indices):
    b, c, d = x.shape
    (n_idx,) = indices.shape
    x_flat = x.reshape(b * c, d)
    out_flat = _connector_sc(x_flat, indices, n_rows=b * c, n_idx=n_idx, d=d)
    return out_flat.reshape(b, n_idx, d)


# speedup vs baseline: 16.1206x; 16.1206x over previous
import functools

import jax
import jax.numpy as jnp
from jax import lax
from jax.experimental import pallas as pl
from jax.experimental.pallas import tpu as pltpu
from jax.experimental.pallas import tpu_sc as plsc

_LANES = 16
_CHUNK = 8
_NBUF = 2
_K_SC = 16


def _sc_gather(x_flat, indices, *, n_rows, n_idx, d, n_batches, rows_per_batch):
    num_workers = 32
    w_per_batch = num_workers // n_batches
    idx_per_w = n_idx // w_per_batch
    n_chunks = idx_per_w // _CHUNK
    assert n_chunks % _NBUF == 0 and n_chunks >= 2 * _NBUF
    mesh = plsc.VectorSubcoreMesh(core_axis_name="c", subcore_axis_name="s")

    @functools.partial(
        pl.kernel,
        mesh=mesh,
        out_type=jax.ShapeDtypeStruct((n_batches * n_idx, d), jnp.float32),
        scratch_types=[
            pltpu.VMEM((n_idx,), jnp.int32),
            pltpu.VMEM((_NBUF, _CHUNK, d), jnp.float32),
            pltpu.SemaphoreType.DMA,
            pltpu.SemaphoreType.DMA,
        ],
    )
    def k(x_hbm, idx_hbm, out_hbm, idx_v, rows_v, gsem, ssem):
        wid = lax.axis_index("s") * 2 + lax.axis_index("c")
        batch = wid // w_per_batch
        part = wid % w_per_batch
        pltpu.sync_copy(idx_hbm, idx_v)
        row_base = batch * rows_per_batch
        for i in range(n_idx // _LANES):
            sl = pl.ds(i * _LANES, _LANES)
            idx_v[sl] = idx_v[sl] + row_base

        idx_off = part * idx_per_w
        out_base = batch * n_idx + idx_off

        def gather(c, buf):
            return pltpu.async_copy(
                x_hbm.at[idx_v.at[pl.ds(idx_off + c * _CHUNK, _CHUNK)]],
                rows_v.at[buf],
                gsem,
            )

        def scatter(c, buf):
            return pltpu.async_copy(
                rows_v.at[buf],
                out_hbm.at[pl.ds(out_base + c * _CHUNK, _CHUNK)],
                ssem,
            )

        def wait_gather(buf):
            pltpu.make_async_copy(
                x_hbm.at[pl.ds(0, _CHUNK)], rows_v.at[buf], gsem
            ).wait()

        def wait_scatter(buf):
            pltpu.make_async_copy(
                rows_v.at[buf], out_hbm.at[pl.ds(out_base, _CHUNK)], ssem
            ).wait()

        for b in range(_NBUF):
            gather(b, b)

        @pl.loop(0, n_chunks - _NBUF, step=_NBUF)
        def _(c0):
            for b in range(_NBUF):
                c = c0 + b
                wait_gather(b)
                scatter(c, b)
                wait_scatter(b)
                gather(c + _NBUF, b)

        for b in range(_NBUF):
            c = n_chunks - _NBUF + b
            wait_gather(b)
            scatter(c, b)
        for b in range(_NBUF):
            wait_scatter(b)

    return k(x_flat, indices)


def _tc_gather(x, indices, *, n_batches, k_sc, n_idx, d, n_ch):
    def body(idx_ref, x_ref, o_ref):
        for j in range(n_idx):
            o_ref[0, j, :] = x_ref[0, idx_ref[j], :]

    return pl.pallas_call(
        body,
        grid_spec=pltpu.PrefetchScalarGridSpec(
            num_scalar_prefetch=1,
            grid=(n_batches - k_sc,),
            in_specs=[
                pl.BlockSpec((1, n_ch, d), lambda b, idx_ref: (b + k_sc, 0, 0))
            ],
            out_specs=pl.BlockSpec(
                (1, n_idx, d), lambda b, idx_ref: (b + k_sc, 0, 0)
            ),
        ),
        out_shape=jax.ShapeDtypeStruct((n_batches, n_idx, d), jnp.float32),
    )(indices, x)


def _tc_merge(out_sc_flat, out_full, *, k_sc, n_idx, d):
    def body(sc_ref, full_ref, o_ref):
        o_ref[...] = sc_ref[...]

    return pl.pallas_call(
        body,
        grid=(k_sc,),
        in_specs=[
            pl.BlockSpec((1, n_idx, d), lambda b: (b, 0, 0)),
            pl.BlockSpec((1, 8, 128), lambda b: (b, 0, 0)),
        ],
        out_specs=pl.BlockSpec((1, n_idx, d), lambda b: (b, 0, 0)),
        out_shape=jax.ShapeDtypeStruct(out_full.shape, out_full.dtype),
        input_output_aliases={1: 0},
    )(out_sc_flat.reshape(k_sc, n_idx, d), out_full)


def kernel(x, indices):
    nb, nch, d = x.shape
    (n_idx,) = indices.shape
    x_flat = x.reshape(nb * nch, d)
    out_sc = _sc_gather(
        x_flat,
        indices,
        n_rows=nb * nch,
        n_idx=n_idx,
        d=d,
        n_batches=_K_SC,
        rows_per_batch=nch,
    )
    out_full = _tc_gather(
        x, indices, n_batches=nb, k_sc=_K_SC, n_idx=n_idx, d=d, n_ch=nch
    )
    return _tc_merge(out_sc, out_full, k_sc=_K_SC, n_idx=n_idx, d=d)


# speedup vs baseline: 23.5699x; 1.4621x over previous
import functools

import jax
import jax.numpy as jnp
from jax import lax
from jax.experimental import pallas as pl
from jax.experimental.pallas import tpu as pltpu
from jax.experimental.pallas import tpu_sc as plsc

_LANES = 16
_CHUNK = 8
_NBUF = 3


def _connector_sc(x_flat, indices, *, n_rows, n_idx, d):
    num_workers = 32
    rows_per_batch = n_rows // num_workers
    n_chunks = n_idx // _CHUNK
    assert (n_chunks - 2) % _NBUF == 0 and n_chunks >= _NBUF
    mesh = plsc.VectorSubcoreMesh(core_axis_name="c", subcore_axis_name="s")

    @functools.partial(
        pl.kernel,
        mesh=mesh,
        out_type=jax.ShapeDtypeStruct((num_workers * n_idx, d), jnp.float32),
        scratch_types=[
            pltpu.VMEM((n_idx,), jnp.int32),
            pltpu.VMEM((_NBUF, _CHUNK, d), jnp.float32),
            pltpu.SemaphoreType.DMA,
            pltpu.SemaphoreType.DMA,
        ],
    )
    def k(x_hbm, idx_hbm, out_hbm, idx_v, rows_v, gsem, ssem):
        wid = lax.axis_index("s") * 2 + lax.axis_index("c")
        pltpu.sync_copy(idx_hbm, idx_v)
        row_base = wid * rows_per_batch
        for i in range(n_idx // _LANES):
            sl = pl.ds(i * _LANES, _LANES)
            idx_v[sl] = idx_v[sl] + row_base

        out_base = wid * n_idx

        def gather(c, buf):
            return pltpu.async_copy(
                x_hbm.at[idx_v.at[pl.ds(c * _CHUNK, _CHUNK)]],
                rows_v.at[buf],
                gsem,
            )

        def scatter(c, buf):
            return pltpu.async_copy(
                rows_v.at[buf],
                out_hbm.at[pl.ds(out_base + c * _CHUNK, _CHUNK)],
                ssem,
            )

        def wait_gather(buf):
            pltpu.make_async_copy(x_hbm.at[pl.ds(0, _CHUNK)], rows_v.at[buf], gsem).wait()

        def wait_scatter(buf):
            pltpu.make_async_copy(
                rows_v.at[buf], out_hbm.at[pl.ds(out_base, _CHUNK)], ssem
            ).wait()

        gather(0, 0)
        gather(1, 1)

        @pl.loop(0, n_chunks - 2, step=_NBUF)
        def _(c0):
            for b in range(_NBUF):
                c = c0 + b

                @pl.when(c >= 1)
                def _():
                    wait_scatter(0)

                gather(c + 2, (b + 2) % _NBUF)
                wait_gather(b)
                scatter(c, b)

        for c in (n_chunks - 2, n_chunks - 1):
            b = c % _NBUF
            wait_gather(b)
            scatter(c, b)
        for _ in range(_NBUF):
            wait_scatter(0)

    return k(x_flat, indices)


def kernel(x, indices):
    b, c, d = x.shape
    (n_idx,) = indices.shape
    x_flat = x.reshape(b * c, d)
    out_flat = _connector_sc(x_flat, indices, n_rows=b * c, n_idx=n_idx, d=d)
    return out_flat.reshape(b, n_idx, d)


# speedup vs baseline: 23.6068x; 1.0016x over previous
import functools

import jax
import jax.numpy as jnp
from jax import lax
from jax.experimental import pallas as pl
from jax.experimental.pallas import tpu as pltpu
from jax.experimental.pallas import tpu_sc as plsc

_LANES = 16
_CHUNK = 8
_NBUF = 2


def _connector_sc(x_flat, indices, *, n_rows, n_idx, d):
    num_workers = 32
    rows_per_batch = n_rows // num_workers
    n_chunks = n_idx // _CHUNK
    assert n_chunks % _NBUF == 0 and n_chunks >= 2 * _NBUF
    mesh = plsc.VectorSubcoreMesh(core_axis_name="c", subcore_axis_name="s")

    @functools.partial(
        pl.kernel,
        mesh=mesh,
        out_type=jax.ShapeDtypeStruct((num_workers * n_idx, d), jnp.float32),
        scratch_types=[
            pltpu.VMEM((n_idx,), jnp.int32),
            pltpu.VMEM((_NBUF, _CHUNK, d), jnp.float32),
            pltpu.SemaphoreType.DMA,
            pltpu.SemaphoreType.DMA,
        ],
    )
    def k(x_hbm, idx_hbm, out_hbm, idx_v, rows_v, gsem, ssem):
        wid = lax.axis_index("s") * 2 + lax.axis_index("c")
        pltpu.sync_copy(idx_hbm, idx_v)
        row_base = wid * rows_per_batch
        for i in range(n_idx // _LANES):
            sl = pl.ds(i * _LANES, _LANES)
            idx_v[sl] = idx_v[sl] + row_base

        out_base = wid * n_idx

        def gather(c, buf):
            return pltpu.async_copy(
                x_hbm.at[idx_v.at[pl.ds(c * _CHUNK, _CHUNK)]],
                rows_v.at[buf],
                gsem,
            )

        def scatter(c, buf):
            return pltpu.async_copy(
                rows_v.at[buf],
                out_hbm.at[pl.ds(out_base + c * _CHUNK, _CHUNK)],
                ssem,
            )

        def wait_gather(buf):
            pltpu.make_async_copy(x_hbm.at[pl.ds(0, _CHUNK)], rows_v.at[buf], gsem).wait()

        def wait_scatter(buf):
            pltpu.make_async_copy(
                rows_v.at[buf], out_hbm.at[pl.ds(out_base, _CHUNK)], ssem
            ).wait()

        for b in range(_NBUF):
            gather(b, b)

        @pl.loop(0, n_chunks - _NBUF, step=_NBUF)
        def _(c0):
            for b in range(_NBUF):
                c = c0 + b
                wait_gather(b)
                scatter(c, b)
                wait_scatter(b)
                gather(c + _NBUF, b)

        for b in range(_NBUF):
            c = n_chunks - _NBUF + b
            wait_gather(b)
            scatter(c, b)
        for b in range(_NBUF):
            wait_scatter(b)

    return k(x_flat, indices)


def kernel(x, indices):
    b, c, d = x.shape
    (n_idx,) = indices.shape
    x_flat = x.reshape(b * c, d)
    out_flat = _connector_sc(x_flat, indices, n_rows=b * c, n_idx=n_idx, d=d)
    return out_flat.reshape(b, n_idx, d)
